# Initial kernel scaffold; baseline (speedup 1.0000x reference)
import jax
import jax.numpy as jnp
from jax.experimental import pallas as pl


def kernel(z_flowed, target_attrs, train_embeddings, train_attributes):
    raise NotImplementedError("write your pallas kernel here")


# speedup vs baseline: 115.7340x; 115.7340x over previous
import jax
import jax.numpy as jnp
from jax.experimental import pallas as pl
from jax.experimental.pallas import tpu as pltpu

_K = 8
_TILE = 2048
_N_TRAIN = 100000
_N_PAD = 100352


def _loss_kernel(z_ref, tattr_ref, emb_ref, attr_ref, out_ref, top8, cnt):
    i = pl.program_id(0)
    nsteps = pl.num_programs(0)
    batch = z_ref.shape[0]
    tile = emb_ref.shape[0]
    nattr = tattr_ref.shape[1]

    @pl.when(i == 0)
    def _init():
        top8[...] = jnp.full_like(top8[...], -jnp.inf)
        cnt[...] = jnp.zeros_like(cnt[...])

    z = z_ref[...]
    zn = z / jnp.maximum(
        jnp.sqrt(jnp.sum(z * z, axis=1, keepdims=True)), 1e-12)

    emb = emb_ref[...]
    embn = emb / jnp.maximum(
        jnp.sqrt(jnp.sum(emb * emb, axis=1, keepdims=True)), 1e-12)
    sims = jax.lax.dot_general(
        zn, embn, (((1,), (1,)), ((), ())),
        preferred_element_type=jnp.float32,
        precision=jax.lax.Precision.HIGHEST)

    pw_i = jax.lax.shift_left(
        jnp.ones((1, nattr), jnp.int32),
        jax.lax.broadcasted_iota(jnp.int32, (1, nattr), 1))
    pw = pw_i.astype(jnp.float32)
    tcode = jax.lax.dot_general(
        tattr_ref[...].astype(jnp.float32), pw, (((1,), (1,)), ((), ())),
        preferred_element_type=jnp.float32).astype(jnp.int32)
    rcode = jax.lax.dot_general(
        pw, attr_ref[...].astype(jnp.float32), (((1,), (1,)), ((), ())),
        preferred_element_type=jnp.float32).astype(jnp.int32)
    x = jax.lax.bitwise_xor(tcode, rcode)
    match = jax.lax.bitwise_and(x, x - 1) == 0

    gidx = i * tile + jax.lax.broadcasted_iota(jnp.int32, (1, tile), 1)
    rowvalid = gidx < _N_TRAIN
    maskf = jnp.where(jnp.logical_and(match, rowvalid), 1.0, 0.0)

    cnt[...] += jnp.sum(maskf, axis=1, keepdims=True)

    masked_sim = jnp.where(rowvalid, sims * maskf, -jnp.inf)

    cand = jnp.concatenate([masked_sim, top8[...]], axis=1)
    width = tile + _K
    iota = jax.lax.broadcasted_iota(jnp.int32, (batch, width), 1)
    tops = []
    vals = cand
    for _ in range(_K):
        m = jnp.max(vals, axis=1, keepdims=True)
        first = jnp.min(jnp.where(vals == m, iota, width), axis=1,
                        keepdims=True)
        vals = jnp.where(iota == first, -jnp.inf, vals)
        tops.append(m)
    new_top8 = jnp.concatenate(tops, axis=1)
    top8[...] = new_top8

    @pl.when(i == nsteps - 1)
    def _final():
        s = jnp.sum(new_top8, axis=1, keepdims=True)
        valid = cnt[...] >= jnp.float32(_K)
        loss = jnp.where(valid, (jnp.float32(_K) - s) / jnp.float32(_K),
                         0.0)
        out_ref[...] = jnp.sum(loss, keepdims=True) / jnp.float32(batch)


def kernel(z_flowed, target_attrs, train_embeddings, train_attributes):
    n = train_embeddings.shape[0]
    pad = _N_PAD - n
    emb = jnp.pad(train_embeddings, ((0, pad), (0, 0)))
    attr = jnp.pad(train_attributes, ((0, pad), (0, 0)))
    grid = _N_PAD // _TILE

    out = pl.pallas_call(
        _loss_kernel,
        grid=(grid,),
        in_specs=[
            pl.BlockSpec(z_flowed.shape, lambda i: (0, 0)),
            pl.BlockSpec(target_attrs.shape, lambda i: (0, 0)),
            pl.BlockSpec((_TILE, emb.shape[1]), lambda i: (i, 0)),
            pl.BlockSpec((_TILE, attr.shape[1]), lambda i: (i, 0)),
        ],
        out_specs=pl.BlockSpec((1, 1), lambda i: (0, 0)),
        out_shape=jax.ShapeDtypeStruct((1, 1), jnp.float32),
        scratch_shapes=[
            pltpu.VMEM((z_flowed.shape[0], _K), jnp.float32),
            pltpu.VMEM((z_flowed.shape[0], 1), jnp.float32),
        ],
    )(z_flowed, target_attrs, emb, attr)
    return out[0, 0]


# speedup vs baseline: 123.4197x; 1.0664x over previous
import jax
import jax.numpy as jnp
from jax.experimental import pallas as pl
from jax.experimental.pallas import tpu as pltpu

_K = 8
_TILE = 2048
_N_TRAIN = 100000
_N_PAD = 100352


def _loss_kernel(z_ref, tattr_ref, emb_ref, attr_ref, out_ref,
                 top8, cnt, zn_s, tcode_s):
    i = pl.program_id(0)
    nsteps = pl.num_programs(0)
    batch = z_ref.shape[0]
    tile = emb_ref.shape[0]
    nattr = tattr_ref.shape[1]

    pw_i = jax.lax.shift_left(
        jnp.ones((1, nattr), jnp.int32),
        jax.lax.broadcasted_iota(jnp.int32, (1, nattr), 1))

    @pl.when(i == 0)
    def _init():
        top8[...] = jnp.full_like(top8[...], -jnp.inf)
        cnt[...] = jnp.zeros_like(cnt[...])
        z = z_ref[...]
        zn_s[...] = z / jnp.maximum(
            jnp.sqrt(jnp.sum(z * z, axis=1, keepdims=True)), 1e-12)
        tcode_s[...] = jax.lax.dot_general(
            tattr_ref[...].astype(jnp.float32), pw_i.astype(jnp.float32),
            (((1,), (1,)), ((), ())),
            preferred_element_type=jnp.float32).astype(jnp.int32)

    emb = emb_ref[...]
    embn = emb / jnp.maximum(
        jnp.sqrt(jnp.sum(emb * emb, axis=1, keepdims=True)), 1e-12)
    sims = jax.lax.dot_general(
        zn_s[...], embn, (((1,), (1,)), ((), ())),
        preferred_element_type=jnp.float32,
        precision=jax.lax.Precision.HIGHEST)

    rcode = jax.lax.dot_general(
        pw_i.astype(jnp.float32), attr_ref[...].astype(jnp.float32),
        (((1,), (1,)), ((), ())),
        preferred_element_type=jnp.float32).astype(jnp.int32)
    x = jax.lax.bitwise_xor(tcode_s[...], rcode)
    match = jax.lax.bitwise_and(x, x - 1) == 0

    gidx = i * tile + jax.lax.broadcasted_iota(jnp.int32, (1, tile), 1)
    rowvalid = gidx < _N_TRAIN
    maskf = jnp.where(jnp.logical_and(match, rowvalid), 1.0, 0.0)

    cnt[...] += jnp.sum(maskf, axis=1, keepdims=True)

    masked_sim = jnp.where(rowvalid, sims * maskf, -jnp.inf)

    vals = jnp.concatenate([masked_sim, top8[...]], axis=1)
    ms = []
    cs = []
    for _ in range(_K):
        m = jnp.max(vals, axis=1, keepdims=True)
        eq = vals == m
        cs.append(jnp.sum(eq.astype(jnp.float32), axis=1, keepdims=True))
        vals = jnp.where(eq, -jnp.inf, vals)
        ms.append(m)

    ranks = jax.lax.broadcasted_iota(jnp.int32, (1, _K), 1).astype(
        jnp.float32)
    new_top8 = jnp.full((batch, _K), -jnp.inf, jnp.float32)
    cum = jnp.zeros((batch, 1), jnp.float32)
    for j in range(_K):
        lo = cum
        cum = cum + cs[j]
        sel = jnp.logical_and(lo <= ranks, ranks < cum)
        new_top8 = jnp.where(sel, ms[j], new_top8)
    top8[...] = new_top8

    @pl.when(i == nsteps - 1)
    def _final():
        s = jnp.sum(new_top8, axis=1, keepdims=True)
        valid = cnt[...] >= jnp.float32(_K)
        loss = jnp.where(valid, (jnp.float32(_K) - s) / jnp.float32(_K),
                         0.0)
        out_ref[...] = jnp.sum(loss, keepdims=True) / jnp.float32(batch)


def kernel(z_flowed, target_attrs, train_embeddings, train_attributes):
    n = train_embeddings.shape[0]
    pad = _N_PAD - n
    emb = jnp.pad(train_embeddings, ((0, pad), (0, 0)))
    attr = jnp.pad(train_attributes, ((0, pad), (0, 0)))
    grid = _N_PAD // _TILE

    out = pl.pallas_call(
        _loss_kernel,
        grid=(grid,),
        in_specs=[
            pl.BlockSpec(z_flowed.shape, lambda i: (0, 0)),
            pl.BlockSpec(target_attrs.shape, lambda i: (0, 0)),
            pl.BlockSpec((_TILE, emb.shape[1]), lambda i: (i, 0)),
            pl.BlockSpec((_TILE, attr.shape[1]), lambda i: (i, 0)),
        ],
        out_specs=pl.BlockSpec((1, 1), lambda i: (0, 0)),
        out_shape=jax.ShapeDtypeStruct((1, 1), jnp.float32),
        scratch_shapes=[
            pltpu.VMEM((z_flowed.shape[0], _K), jnp.float32),
            pltpu.VMEM((z_flowed.shape[0], 1), jnp.float32),
            pltpu.VMEM(z_flowed.shape, jnp.float32),
            pltpu.VMEM((z_flowed.shape[0], 1), jnp.int32),
        ],
    )(z_flowed, target_attrs, emb, attr)
    return out[0, 0]


# speedup vs baseline: 222.2309x; 1.8006x over previous
import jax
import jax.numpy as jnp
from jax.experimental import pallas as pl
from jax.experimental.pallas import tpu as pltpu

_K = 8
_TILE = 2048
_N_TRAIN = 100000
_N_PAD = 100352

_SORT8 = [(0, 1), (2, 3), (4, 5), (6, 7),
          (0, 2), (1, 3), (4, 6), (5, 7),
          (1, 2), (5, 6),
          (0, 4), (1, 5), (2, 6), (3, 7),
          (2, 4), (3, 5),
          (1, 2), (3, 4), (5, 6)]

_BITONIC8 = [(0, 4), (1, 5), (2, 6), (3, 7),
             (0, 2), (1, 3), (4, 6), (5, 7),
             (0, 1), (2, 3), (4, 5), (6, 7)]


def _ce(slots, net):
    for i, j in net:
        a, b = slots[i], slots[j]
        slots[i] = jnp.maximum(a, b)
        slots[j] = jnp.minimum(a, b)
    return slots


def _merge_topk(a, b):
    c = [jnp.maximum(a[i], b[_K - 1 - i]) for i in range(_K)]
    return _ce(c, _BITONIC8)


def _loss_kernel(z_ref, tattr_ref, emb_ref, attr_ref, out_ref,
                 top8, cnt, zn_s, tcode_s):
    i = pl.program_id(0)
    nsteps = pl.num_programs(0)
    batch = z_ref.shape[0]
    tile = emb_ref.shape[0]
    nattr = tattr_ref.shape[1]

    pw = jax.lax.shift_left(
        jnp.ones((1, nattr), jnp.int32),
        jax.lax.broadcasted_iota(jnp.int32, (1, nattr), 1)).astype(
            jnp.float32)

    @pl.when(i == 0)
    def _init():
        top8[...] = jnp.full_like(top8[...], -jnp.inf)
        cnt[...] = jnp.zeros_like(cnt[...])
        z = z_ref[...]
        zn_s[...] = z / jnp.maximum(
            jnp.sqrt(jnp.sum(z * z, axis=1, keepdims=True)), 1e-12)
        tcode_s[...] = jax.lax.dot_general(
            pw, tattr_ref[...].astype(jnp.float32),
            (((1,), (1,)), ((), ())),
            preferred_element_type=jnp.float32).astype(jnp.int32)

    emb = emb_ref[...]
    embn = emb / jnp.maximum(
        jnp.sqrt(jnp.sum(emb * emb, axis=1, keepdims=True)), 1e-12)
    sims = jax.lax.dot_general(
        embn, zn_s[...], (((1,), (1,)), ((), ())),
        preferred_element_type=jnp.float32,
        precision=jax.lax.Precision.HIGHEST)

    rcode = jax.lax.dot_general(
        attr_ref[...].astype(jnp.float32), pw, (((1,), (1,)), ((), ())),
        preferred_element_type=jnp.float32).astype(jnp.int32)
    x = jax.lax.bitwise_xor(rcode, tcode_s[...])
    match = jax.lax.bitwise_and(x, x - 1) == 0

    gidx = i * tile + jax.lax.broadcasted_iota(jnp.int32, (tile, 1), 0)
    rowvalid = gidx < _N_TRAIN
    keep = jnp.logical_and(match, rowvalid)
    cnt[...] += jnp.sum(keep.astype(jnp.float32), axis=0, keepdims=True)

    base = jnp.where(rowvalid, 0.0, -jnp.inf)
    masked_sim = jnp.where(keep, sims, base)

    seg = tile // _K
    slots = [masked_sim[d * seg:(d + 1) * seg, :] for d in range(_K)]
    slots = _ce(slots, _SORT8)
    w = seg // 2
    while w >= 1:
        lo = [s[:w, :] for s in slots]
        hi = [s[w:2 * w, :] for s in slots]
        slots = _merge_topk(lo, hi)
        w //= 2

    state = [top8[d:d + 1, :] for d in range(_K)]
    merged = _merge_topk(slots, state)
    for d in range(_K):
        top8[d:d + 1, :] = merged[d]

    @pl.when(i == nsteps - 1)
    def _final():
        s = merged[0]
        for d in range(1, _K):
            s = s + merged[d]
        valid = cnt[...] >= jnp.float32(_K)
        loss = jnp.where(valid, (jnp.float32(_K) - s) / jnp.float32(_K),
                         0.0)
        out_ref[...] = jnp.sum(loss, keepdims=True) / jnp.float32(batch)


def kernel(z_flowed, target_attrs, train_embeddings, train_attributes):
    n = train_embeddings.shape[0]
    pad = _N_PAD - n
    emb = jnp.pad(train_embeddings, ((0, pad), (0, 0)))
    attr = jnp.pad(train_attributes, ((0, pad), (0, 0)))
    grid = _N_PAD // _TILE
    batch = z_flowed.shape[0]

    out = pl.pallas_call(
        _loss_kernel,
        grid=(grid,),
        in_specs=[
            pl.BlockSpec(z_flowed.shape, lambda i: (0, 0)),
            pl.BlockSpec(target_attrs.shape, lambda i: (0, 0)),
            pl.BlockSpec((_TILE, emb.shape[1]), lambda i: (i, 0)),
            pl.BlockSpec((_TILE, attr.shape[1]), lambda i: (i, 0)),
        ],
        out_specs=pl.BlockSpec((1, 1), lambda i: (0, 0)),
        out_shape=jax.ShapeDtypeStruct((1, 1), jnp.float32),
        scratch_shapes=[
            pltpu.VMEM((_K, batch), jnp.float32),
            pltpu.VMEM((1, batch), jnp.float32),
            pltpu.VMEM(z_flowed.shape, jnp.float32),
            pltpu.VMEM((1, batch), jnp.int32),
        ],
    )(z_flowed, target_attrs, emb, attr)
    return out[0, 0]


# speedup vs baseline: 322.1221x; 1.4495x over previous
import jax
import jax.numpy as jnp
from jax.experimental import pallas as pl
from jax.experimental.pallas import tpu as pltpu

_K = 8
_TILE = 4096
_N_TRAIN = 100000
_N_PAD = 102400

_SORT8 = [(0, 1), (2, 3), (4, 5), (6, 7),
          (0, 2), (1, 3), (4, 6), (5, 7),
          (1, 2), (5, 6),
          (0, 4), (1, 5), (2, 6), (3, 7),
          (2, 4), (3, 5),
          (1, 2), (3, 4), (5, 6)]

_BITONIC8 = [(0, 4), (1, 5), (2, 6), (3, 7),
             (0, 2), (1, 3), (4, 6), (5, 7),
             (0, 1), (2, 3), (4, 5), (6, 7)]


def _ce(slots, net):
    for i, j in net:
        a, b = slots[i], slots[j]
        slots[i] = jnp.maximum(a, b)
        slots[j] = jnp.minimum(a, b)
    return slots


def _merge_topk(a, b):
    c = [jnp.maximum(a[i], b[_K - 1 - i]) for i in range(_K)]
    return _ce(c, _BITONIC8)


def _loss_kernel(z_ref, tattr_ref, emb_ref, attr_ref, out_ref,
                 top8, cnt, zn_s, tcode_s):
    i = pl.program_id(0)
    nsteps = pl.num_programs(0)
    batch = z_ref.shape[0]
    tile = emb_ref.shape[0]
    nattr = tattr_ref.shape[1]

    pw = jax.lax.shift_left(
        jnp.ones((1, nattr), jnp.int32),
        jax.lax.broadcasted_iota(jnp.int32, (1, nattr), 1)).astype(
            jnp.float32)

    @pl.when(i == 0)
    def _init():
        top8[...] = jnp.full_like(top8[...], -jnp.inf)
        cnt[...] = jnp.zeros_like(cnt[...])
        z = z_ref[...]
        zn_s[...] = z / jnp.maximum(
            jnp.sqrt(jnp.sum(z * z, axis=1, keepdims=True)), 1e-12)
        tcode_s[...] = jax.lax.dot_general(
            pw, tattr_ref[...].astype(jnp.float32),
            (((1,), (1,)), ((), ())),
            preferred_element_type=jnp.float32).astype(jnp.int32)

    emb = emb_ref[...]
    embn = emb / jnp.maximum(
        jnp.sqrt(jnp.sum(emb * emb, axis=1, keepdims=True)), 1e-12)
    sims = jax.lax.dot_general(
        embn, zn_s[...], (((1,), (1,)), ((), ())),
        preferred_element_type=jnp.float32)

    rcode = jax.lax.dot_general(
        attr_ref[...].astype(jnp.float32), pw, (((1,), (1,)), ((), ())),
        preferred_element_type=jnp.float32).astype(jnp.int32)
    x = jax.lax.bitwise_xor(rcode, tcode_s[...])
    match = jax.lax.bitwise_and(x, x - 1) == 0

    gidx = i * tile + jax.lax.broadcasted_iota(jnp.int32, (tile, 1), 0)
    rowvalid = gidx < _N_TRAIN
    keep = jnp.logical_and(match, rowvalid)
    cnt[...] += jnp.sum(keep.astype(jnp.float32), axis=0, keepdims=True)

    base = jnp.where(rowvalid, 0.0, -jnp.inf)
    masked_sim = jnp.where(keep, sims, base)

    seg = tile // _K
    slots = [masked_sim[d * seg:(d + 1) * seg, :] for d in range(_K)]
    slots = _ce(slots, _SORT8)
    w = seg // 2
    while w >= 1:
        lo = [s[:w, :] for s in slots]
        hi = [s[w:2 * w, :] for s in slots]
        slots = _merge_topk(lo, hi)
        w //= 2

    state = [top8[d:d + 1, :] for d in range(_K)]
    merged = _merge_topk(slots, state)
    for d in range(_K):
        top8[d:d + 1, :] = merged[d]

    @pl.when(i == nsteps - 1)
    def _final():
        s = merged[0]
        for d in range(1, _K):
            s = s + merged[d]
        valid = cnt[...] >= jnp.float32(_K)
        loss = jnp.where(valid, (jnp.float32(_K) - s) / jnp.float32(_K),
                         0.0)
        out_ref[...] = jnp.sum(loss, keepdims=True) / jnp.float32(batch)


def kernel(z_flowed, target_attrs, train_embeddings, train_attributes):
    n = train_embeddings.shape[0]
    pad = _N_PAD - n
    emb = jnp.pad(train_embeddings, ((0, pad), (0, 0)))
    attr = jnp.pad(train_attributes, ((0, pad), (0, 0)))
    grid = _N_PAD // _TILE
    batch = z_flowed.shape[0]

    out = pl.pallas_call(
        _loss_kernel,
        grid=(grid,),
        in_specs=[
            pl.BlockSpec(z_flowed.shape, lambda i: (0, 0)),
            pl.BlockSpec(target_attrs.shape, lambda i: (0, 0)),
            pl.BlockSpec((_TILE, emb.shape[1]), lambda i: (i, 0)),
            pl.BlockSpec((_TILE, attr.shape[1]), lambda i: (i, 0)),
        ],
        out_specs=pl.BlockSpec((1, 1), lambda i: (0, 0)),
        out_shape=jax.ShapeDtypeStruct((1, 1), jnp.float32),
        scratch_shapes=[
            pltpu.VMEM((_K, batch), jnp.float32),
            pltpu.VMEM((1, batch), jnp.float32),
            pltpu.VMEM(z_flowed.shape, jnp.float32),
            pltpu.VMEM((1, batch), jnp.int32),
        ],
    )(z_flowed, target_attrs, emb, attr)
    return out[0, 0]


# speedup vs baseline: 359.6320x; 1.1164x over previous
import jax
import jax.numpy as jnp
from jax.experimental import pallas as pl
from jax.experimental.pallas import tpu as pltpu

_K = 8
_TILE = 4096
_N_TRAIN = 100000
_N_PAD = 102400

_SORT8 = [(0, 1), (2, 3), (4, 5), (6, 7),
          (0, 2), (1, 3), (4, 6), (5, 7),
          (1, 2), (5, 6),
          (0, 4), (1, 5), (2, 6), (3, 7),
          (2, 4), (3, 5),
          (1, 2), (3, 4), (5, 6)]

_BITONIC8 = [(0, 4), (1, 5), (2, 6), (3, 7),
             (0, 2), (1, 3), (4, 6), (5, 7),
             (0, 1), (2, 3), (4, 5), (6, 7)]


def _ce(slots, net):
    for i, j in net:
        a, b = slots[i], slots[j]
        slots[i] = jnp.maximum(a, b)
        slots[j] = jnp.minimum(a, b)
    return slots


def _merge_topk(a, b):
    c = [jnp.maximum(a[i], b[_K - 1 - i]) for i in range(_K)]
    return _ce(c, _BITONIC8)


def _loss_kernel(z_ref, tattr_ref, emb_ref, attr_ref, out_ref,
                 top8, cnt, zn_s, tsign_s):
    i = pl.program_id(0)
    nsteps = pl.num_programs(0)
    batch = z_ref.shape[0]
    tile = emb_ref.shape[0]

    @pl.when(i == 0)
    def _init():
        top8[...] = jnp.full_like(top8[...], -jnp.inf)
        cnt[...] = jnp.zeros_like(cnt[...])
        z = z_ref[...]
        zn_s[...] = z / jnp.maximum(
            jnp.sqrt(jnp.sum(z * z, axis=1, keepdims=True)), 1e-12)
        tsign_s[...] = 2.0 * tattr_ref[...].astype(jnp.float32) - 1.0

    emb = emb_ref[...]
    sq = jnp.sum(emb * emb, axis=1, keepdims=True)
    embn = emb * jax.lax.rsqrt(jnp.maximum(sq, 1e-24))
    sims = jax.lax.dot_general(
        embn, zn_s[...], (((1,), (1,)), ((), ())),
        preferred_element_type=jnp.float32)

    gidx = i * tile + jax.lax.broadcasted_iota(jnp.int32, (tile, 1), 0)
    rowvalid = gidx < _N_TRAIN
    asign = jnp.where(rowvalid,
                      2.0 * attr_ref[...].astype(jnp.float32) - 1.0, 0.0)
    s_match = jax.lax.dot_general(
        asign, tsign_s[...], (((1,), (1,)), ((), ())),
        preferred_element_type=jnp.float32)
    keep = s_match >= 5.5
    cnt[...] += jnp.sum(keep.astype(jnp.float32), axis=0, keepdims=True)

    base = jnp.where(rowvalid, 0.0, -jnp.inf)
    masked_sim = jnp.where(keep, sims, base)

    seg = tile // _K
    slots = [masked_sim[d * seg:(d + 1) * seg, :] for d in range(_K)]
    slots = _ce(slots, _SORT8)
    w = seg // 2
    while w >= 1:
        lo = [s[:w, :] for s in slots]
        hi = [s[w:2 * w, :] for s in slots]
        slots = _merge_topk(lo, hi)
        w //= 2

    state = [top8[d:d + 1, :] for d in range(_K)]
    merged = _merge_topk(slots, state)
    for d in range(_K):
        top8[d:d + 1, :] = merged[d]

    @pl.when(i == nsteps - 1)
    def _final():
        s = merged[0]
        for d in range(1, _K):
            s = s + merged[d]
        valid = cnt[...] >= jnp.float32(_K)
        loss = jnp.where(valid, (jnp.float32(_K) - s) / jnp.float32(_K),
                         0.0)
        out_ref[...] = jnp.sum(loss, keepdims=True) / jnp.float32(batch)


def kernel(z_flowed, target_attrs, train_embeddings, train_attributes):
    n = train_embeddings.shape[0]
    pad = _N_PAD - n
    emb = jnp.pad(train_embeddings, ((0, pad), (0, 0)))
    attr = jnp.pad(train_attributes, ((0, pad), (0, 0)))
    grid = _N_PAD // _TILE
    batch = z_flowed.shape[0]

    out = pl.pallas_call(
        _loss_kernel,
        grid=(grid,),
        in_specs=[
            pl.BlockSpec(z_flowed.shape, lambda i: (0, 0)),
            pl.BlockSpec(target_attrs.shape, lambda i: (0, 0)),
            pl.BlockSpec((_TILE, emb.shape[1]), lambda i: (i, 0)),
            pl.BlockSpec((_TILE, attr.shape[1]), lambda i: (i, 0)),
        ],
        out_specs=pl.BlockSpec((1, 1), lambda i: (0, 0)),
        out_shape=jax.ShapeDtypeStruct((1, 1), jnp.float32),
        scratch_shapes=[
            pltpu.VMEM((_K, batch), jnp.float32),
            pltpu.VMEM((1, batch), jnp.float32),
            pltpu.VMEM(z_flowed.shape, jnp.float32),
            pltpu.VMEM(target_attrs.shape, jnp.float32),
        ],
    )(z_flowed, target_attrs, emb, attr)
    return out[0, 0]
